# baseline (speedup 1.0000x reference)
import functools

import jax
import jax.numpy as jnp
from jax import lax
from jax.experimental import pallas as pl
from jax.experimental.pallas import tpu as pltpu, tpu_sc as plsc

_DIM = 768
_N_EXPERTS = 8
_TOP_K = 2
_BLOCK = 4096

_NC = 2
_NS = 16
_L = 16


def _scores_body(x_ref, w_ref, s_ref):
    s_ref[...] = jax.lax.dot_general(
        w_ref[...], x_ref[...], (((1,), (1,)), ((), ())),
        preferred_element_type=jnp.float32,
    )


def _route_body(tok_per_w, s_hbm, c_hbm, i_hbm, s_v, c_v, i_v):
    wid = lax.axis_index("s") * _NC + lax.axis_index("c")
    base = wid * tok_per_w
    pltpu.sync_copy(s_hbm.at[:, pl.ds(base, tok_per_w)], s_v)

    def group(g, carry):
        off = g * _L
        ss = [s_v[e, pl.ds(off, _L)] for e in range(_N_EXPERTS)]
        m1 = ss[0]
        i1 = jnp.zeros((_L,), jnp.int32)
        m2 = jnp.full((_L,), -jnp.inf, jnp.float32)
        i2 = jnp.zeros((_L,), jnp.int32)
        for e in range(1, _N_EXPERTS):
            se = ss[e]
            gt1 = se > m1
            gt2 = se > m2
            m2n = jnp.where(gt1, m1, jnp.where(gt2, se, m2))
            i2n = jnp.where(gt1, i1, jnp.where(gt2, jnp.int32(e), i2))
            m1 = jnp.where(gt1, se, m1)
            i1 = jnp.where(gt1, jnp.int32(e), i1)
            m2 = m2n
            i2 = i2n
        z = jnp.exp(ss[0] - m1)
        for e in range(1, _N_EXPERTS):
            z = z + jnp.exp(ss[e] - m1)
        c_v[0, pl.ds(off, _L)] = 1.0 / z
        c_v[1, pl.ds(off, _L)] = jnp.exp(m2 - m1) / z
        i_v[0, pl.ds(off, _L)] = i1
        i_v[1, pl.ds(off, _L)] = i2
        return carry

    lax.fori_loop(0, tok_per_w // _L, group, 0)
    pltpu.sync_copy(c_v, c_hbm.at[:, pl.ds(base, tok_per_w)])
    pltpu.sync_copy(i_v, i_hbm.at[:, pl.ds(base, tok_per_w)])


@jax.jit
def kernel(x, W):
    tokens = x.shape[0]
    scores_t = pl.pallas_call(
        _scores_body,
        grid=(tokens // _BLOCK,),
        in_specs=[
            pl.BlockSpec((_BLOCK, _DIM), lambda i: (i, 0)),
            pl.BlockSpec((_N_EXPERTS, _DIM), lambda i: (0, 0)),
        ],
        out_specs=pl.BlockSpec((_N_EXPERTS, _BLOCK), lambda i: (0, i)),
        out_shape=jax.ShapeDtypeStruct((_N_EXPERTS, tokens), jnp.float32),
    )(x, W)

    tok_per_w = tokens // (_NC * _NS)
    mesh = plsc.VectorSubcoreMesh(core_axis_name="c", subcore_axis_name="s")
    c_t, idx_t = pl.kernel(
        functools.partial(_route_body, tok_per_w),
        out_type=[
            jax.ShapeDtypeStruct((_TOP_K, tokens), jnp.float32),
            jax.ShapeDtypeStruct((_TOP_K, tokens), jnp.int32),
        ],
        mesh=mesh,
        scratch_types=[
            pltpu.VMEM((_N_EXPERTS, tok_per_w), jnp.float32),
            pltpu.VMEM((_TOP_K, tok_per_w), jnp.float32),
            pltpu.VMEM((_TOP_K, tok_per_w), jnp.int32),
        ],
    )(scores_t)
    return (c_t.T, idx_t.T)
